# baseline (speedup 1.0000x reference)
import jax
import jax.numpy as jnp
from jax import lax
from jax.experimental import pallas as pl
from jax.experimental.pallas import tpu as pltpu
from jax.experimental.pallas import tpu_sc as plsc

NUM_EMB = 1_000_000
NUM_SUB = 8
CB_SIZE = 256
SUB_DIM = 16
EMB_DIM = NUM_SUB * SUB_DIM

N_TOKENS = 4096 * 50
NC, NS = 2, 16
NW = NC * NS
CHUNK = 256
NIDX = CHUNK // 128
PER_W = N_TOKENS // NW
N_CHUNKS = PER_W // CHUNK
TOTAL_CHUNKS = NW * N_CHUNKS
ROWS = CHUNK * NUM_SUB


def _pq_body(ids_hbm, cb_hbm, q0, q1, out_hbm,
             ids_v, codes_v, fidx_v, out_v, cb_sh, sg, sp, sw):
    qs = (q0, q1)
    cid = lax.axis_index("c")
    sid = lax.axis_index("s")
    wid = sid * NC + cid

    @pl.when(sid == 0)
    def _():
        pltpu.sync_copy(cb_hbm, cb_sh)

    plsc.subcore_barrier()

    iota = lax.iota(jnp.int32, 16)
    lane_div8 = iota // 8
    lane_mod8 = iota % 8
    bias = lane_mod8 * CB_SIZE
    lane_quad = lane_mod8 // 4
    lane_shift = (lane_mod8 % 4) * 8

    def chunk_base(g):
        return ((wid * N_CHUNKS + g) % TOTAL_CHUNKS) * CHUNK

    def prefetch(g, b):
        for k in range(NIDX):
            pltpu.sync_copy(
                ids_hbm.at[pl.ds(chunk_base(g) + k * 128, 128)],
                ids_v[b].at[k])
        for q in range(2):
            for k in range(NIDX):
                pltpu.async_copy(
                    qs[q].at[ids_v[b].at[k]],
                    codes_v[b].at[q, pl.ds(k * 128, 128)], sg[b])

    def expand(g, b):
        for q in range(2):
            for k in range(NIDX):
                pltpu.make_async_copy(
                    qs[q].at[ids_v[b].at[k]],
                    codes_v[b].at[q, pl.ds(k * 128, 128)], sg[b]).wait()
        prefetch(g + 1, 1 - b)
        for i in range(ROWS // 16):
            col = lane_div8 + (2 * i)
            quad = plsc.load_gather(codes_v[b], [lane_quad, col])
            code = (quad >> lane_shift) & 0xFF
            fidx_v[b][i // 8, pl.ds((i % 8) * 16, 16)] = code + bias
        @pl.when(g >= 2)
        def _():
            pltpu.make_async_copy(
                out_v[b],
                out_hbm.at[pl.ds(chunk_base(g - 2) * NUM_SUB, ROWS)],
                sw[b]).wait()
        copies = [
            pltpu.async_copy(cb_sh.at[fidx_v[b].at[j]],
                             out_v[b].at[pl.ds(j * 128, 128)], sp[b])
            for j in range(ROWS // 128)
        ]
        for c in copies:
            c.wait()
        pltpu.async_copy(out_v[b],
                         out_hbm.at[pl.ds(chunk_base(g) * NUM_SUB, ROWS)],
                         sw[b])

    prefetch(0, 0)

    def pair_body(g2, carry):
        expand(2 * g2, 0)
        expand(2 * g2 + 1, 1)
        return carry

    lax.fori_loop(0, N_CHUNKS // 2, pair_body, 0)

    pltpu.make_async_copy(
        out_v[0], out_hbm.at[pl.ds(chunk_base(N_CHUNKS - 2) * NUM_SUB, ROWS)],
        sw[0]).wait()
    pltpu.make_async_copy(
        out_v[1], out_hbm.at[pl.ds(chunk_base(N_CHUNKS - 1) * NUM_SUB, ROWS)],
        sw[1]).wait()
    for q in range(2):
        for k in range(NIDX):
            pltpu.make_async_copy(
                qs[q].at[ids_v[0].at[k]],
                codes_v[0].at[q, pl.ds(k * 128, 128)], sg[0]).wait()


@jax.jit
def _pq_lookup(ids_flat, cb_flat, q0, q1):
    mesh = plsc.VectorSubcoreMesh(core_axis_name="c", subcore_axis_name="s")
    run = pl.kernel(
        _pq_body,
        out_type=jax.ShapeDtypeStruct((N_TOKENS * NUM_SUB, SUB_DIM),
                                      jnp.float32),
        mesh=mesh,
        compiler_params=pltpu.CompilerParams(use_tc_tiling_on_sc=False,
                                             needs_layout_passes=False),
        scratch_types=[
            [pltpu.VMEM((NIDX, 128), jnp.int32)] * 2,
            [pltpu.VMEM((2, CHUNK), jnp.int32)] * 2,
            [pltpu.VMEM((ROWS // 128, 128), jnp.int32)] * 2,
            [pltpu.VMEM((ROWS, SUB_DIM), jnp.float32)] * 2,
            pltpu.VMEM_SHARED((NUM_SUB * CB_SIZE, SUB_DIM), jnp.float32),
            [pltpu.SemaphoreType.DMA] * 2,
            [pltpu.SemaphoreType.DMA] * 2,
            [pltpu.SemaphoreType.DMA] * 2,
        ],
    )
    return run(ids_flat, cb_flat, q0, q1)


def kernel(input_ids, codebooks, codes):
    B, L = input_ids.shape
    ids_t = input_ids.T.reshape(-1).astype(jnp.int32)
    cb_flat = codebooks.reshape(NUM_SUB * CB_SIZE, SUB_DIM)
    w = jnp.zeros((NUM_SUB, 4), jnp.float32)
    w = w.at[0, 0].set(1.0).at[1, 0].set(256.0)
    w = w.at[2, 1].set(1.0).at[3, 1].set(256.0)
    w = w.at[4, 2].set(1.0).at[5, 2].set(256.0)
    w = w.at[6, 3].set(1.0).at[7, 3].set(256.0)
    halves = jax.lax.dot(codes.astype(jnp.float32), w,
                         precision=jax.lax.Precision.HIGHEST)
    h = halves.astype(jnp.int32)
    q0 = h[:, 0] | (h[:, 1] << 16)
    q1 = h[:, 2] | (h[:, 3] << 16)
    out = _pq_lookup(ids_t, cb_flat, q0, q1)
    return jnp.swapaxes(out.reshape(L, B, EMB_DIM), 0, 1)


# speedup vs baseline: 1.0658x; 1.0658x over previous
import jax
import jax.numpy as jnp
from jax import lax
from jax.experimental import pallas as pl
from jax.experimental.pallas import tpu as pltpu
from jax.experimental.pallas import tpu_sc as plsc

NUM_EMB = 1_000_000
NUM_SUB = 8
CB_SIZE = 256
SUB_DIM = 16
EMB_DIM = NUM_SUB * SUB_DIM

N_TOKENS = 4096 * 50
NC, NS = 2, 16
NW = NC * NS
CHUNK = 128
PER_W = N_TOKENS // NW
N_CHUNKS = PER_W // CHUNK
TOTAL_CHUNKS = NW * N_CHUNKS
ROWS = CHUNK * NUM_SUB


def _pq_body(ids_hbm, cb_hbm, q0, q1, out_hbm,
             ids_v, codes_v, fidx_v, out_v, cb_sh, sg, sp, sw):
    qs = (q0, q1)
    cid = lax.axis_index("c")
    sid = lax.axis_index("s")
    wid = sid * NC + cid

    @pl.when(sid == 0)
    def _():
        pltpu.sync_copy(cb_hbm, cb_sh)

    plsc.subcore_barrier()

    iota = lax.iota(jnp.int32, 16)
    lane_div8 = iota // 8
    lane_mod8 = iota % 8
    bias = lane_mod8 * CB_SIZE
    lane_quad = lane_mod8 // 4
    lane_shift = (lane_mod8 % 4) * 8

    def chunk_base(g):
        return ((wid * N_CHUNKS + g) % TOTAL_CHUNKS) * CHUNK

    def prefetch(g, b):
        pltpu.sync_copy(ids_hbm.at[pl.ds(chunk_base(g), CHUNK)], ids_v[b])
        for q in range(2):
            pltpu.async_copy(qs[q].at[ids_v[b]], codes_v[b].at[q], sg[b])

    def expand(g, b):
        for q in range(2):
            pltpu.make_async_copy(qs[q].at[ids_v[b]], codes_v[b].at[q],
                                  sg[b]).wait()
        prefetch(g + 1, 1 - b)
        @pl.when(g >= 2)
        def _():
            pltpu.make_async_copy(
                out_v[b],
                out_hbm.at[pl.ds(chunk_base(g - 2) * NUM_SUB, ROWS)],
                sw[b]).wait()
        copies = []
        for j in range(NUM_SUB):
            for i in range(8 * j, 8 * j + 8):
                col = lane_div8 + (2 * i)
                quad = plsc.load_gather(codes_v[b], [lane_quad, col])
                code = (quad >> lane_shift) & 0xFF
                fidx_v[b][j, pl.ds((i % 8) * 16, 16)] = code + bias
            copies.append(
                pltpu.async_copy(cb_sh.at[fidx_v[b].at[j]],
                                 out_v[b].at[pl.ds(j * CHUNK, CHUNK)], sp[b]))
        for c in copies:
            c.wait()
        pltpu.async_copy(out_v[b],
                         out_hbm.at[pl.ds(chunk_base(g) * NUM_SUB, ROWS)],
                         sw[b])

    prefetch(0, 0)

    def pair_body(g2, carry):
        expand(2 * g2, 0)
        expand(2 * g2 + 1, 1)
        return carry

    lax.fori_loop(0, N_CHUNKS // 2, pair_body, 0)

    pltpu.make_async_copy(
        out_v[0], out_hbm.at[pl.ds(chunk_base(N_CHUNKS - 2) * NUM_SUB, ROWS)],
        sw[0]).wait()
    pltpu.make_async_copy(
        out_v[1], out_hbm.at[pl.ds(chunk_base(N_CHUNKS - 1) * NUM_SUB, ROWS)],
        sw[1]).wait()
    for q in range(2):
        pltpu.make_async_copy(qs[q].at[ids_v[0]], codes_v[0].at[q],
                              sg[0]).wait()


@jax.jit
def _pq_lookup(ids_flat, cb_flat, q0, q1):
    mesh = plsc.VectorSubcoreMesh(core_axis_name="c", subcore_axis_name="s")
    run = pl.kernel(
        _pq_body,
        out_type=jax.ShapeDtypeStruct((N_TOKENS * NUM_SUB, SUB_DIM),
                                      jnp.float32),
        mesh=mesh,
        compiler_params=pltpu.CompilerParams(use_tc_tiling_on_sc=False,
                                             needs_layout_passes=False),
        scratch_types=[
            [pltpu.VMEM((CHUNK,), jnp.int32)] * 2,
            [pltpu.VMEM((2, CHUNK), jnp.int32)] * 2,
            [pltpu.VMEM((NUM_SUB, CHUNK), jnp.int32)] * 2,
            [pltpu.VMEM((ROWS, SUB_DIM), jnp.float32)] * 2,
            pltpu.VMEM_SHARED((NUM_SUB * CB_SIZE, SUB_DIM), jnp.float32),
            [pltpu.SemaphoreType.DMA] * 2,
            [pltpu.SemaphoreType.DMA] * 2,
            [pltpu.SemaphoreType.DMA] * 2,
        ],
    )
    return run(ids_flat, cb_flat, q0, q1)


def kernel(input_ids, codebooks, codes):
    B, L = input_ids.shape
    ids_t = input_ids.T.reshape(-1).astype(jnp.int32)
    cb_flat = codebooks.reshape(NUM_SUB * CB_SIZE, SUB_DIM)
    w = jnp.zeros((NUM_SUB, 4), jnp.float32)
    w = w.at[0, 0].set(1.0).at[1, 0].set(256.0)
    w = w.at[2, 1].set(1.0).at[3, 1].set(256.0)
    w = w.at[4, 2].set(1.0).at[5, 2].set(256.0)
    w = w.at[6, 3].set(1.0).at[7, 3].set(256.0)
    halves = jax.lax.dot(codes.astype(jnp.float32), w,
                         precision=jax.lax.Precision.HIGHEST)
    h = halves.astype(jnp.int32)
    q0 = h[:, 0] | (h[:, 1] << 16)
    q1 = h[:, 2] | (h[:, 3] << 16)
    out = _pq_lookup(ids_t, cb_flat, q0, q1)
    return jnp.swapaxes(out.reshape(L, B, EMB_DIM), 0, 1)
